# baseline (speedup 1.0000x reference)
import jax
import jax.numpy as jnp
from jax import lax
from jax.experimental import pallas as pl
from jax.experimental.pallas import tpu as pltpu
from jax.experimental.pallas import tpu_sc as plsc

N = 10000
E = 320000
D = 128
NC = 2
NS = 16
NW = NC * NS
L = 16
NP = 10240
EPW = E // NW
EPT = E // NS
CH = 80
CH2 = 2000
CHD = 32
RPS = N // NS
STR = NP // NS


def _node_body(h_ref, wfct, wb1t, b1, wb2t, b2, wp, z_ref, b1h_ref, b2h_ref,
               pn_ref):
    hb = h_ref[...]
    z = jnp.dot(hb, wfct[...], preferred_element_type=jnp.float32)
    z_ref[...] = z
    b1h_ref[...] = jnp.dot(hb, wb1t[...],
                           preferred_element_type=jnp.float32) + b1[...][None, :]
    b2h_ref[...] = jnp.dot(hb, wb2t[...],
                           preferred_element_type=jnp.float32) + b2[...][None, :]
    pn_ref[...] = jnp.dot(z, wp[...], preferred_element_type=jnp.float32)


def _edge_body(e_ref, g_ref, wb3t, b3, u2, u2e, ej_ref, pe_ref, qe_ref):
    eb = e_ref[...]
    ej = jax.nn.relu(g_ref[...]
                     + jnp.dot(eb, wb3t[...], preferred_element_type=jnp.float32)
                     + b3[...][None, :]) + eb
    ej_ref[...] = ej
    pe_ref[...] = jnp.dot(ej, u2[...], preferred_element_type=jnp.float32)
    qe_ref[...] = jnp.dot(ej, u2e[...], preferred_element_type=jnp.float32)


def _tail_body(h1_ref, h2_ref, h_ref, wm1t, wct, bmix, gamma, beta, out_ref):
    hs = (jnp.dot(h1_ref[...], wm1t[...], preferred_element_type=jnp.float32)
          + jnp.dot(h2_ref[...], wct[...], preferred_element_type=jnp.float32)
          + 2.0 * bmix[...][None, :])
    mu = jnp.mean(hs, axis=-1, keepdims=True)
    var = jnp.mean((hs - mu) ** 2, axis=-1, keepdims=True)
    hn = (hs - mu) / jnp.sqrt(var + 1e-5) * gamma[...][None, :] + beta[...][None, :]
    out_ref[...] = jax.nn.relu(hn) + h_ref[...]


def _c1_body(src_h, dst_h, b1h_h, b2h_h, pn_h, g_h, t1_h, t2_h, t3_h, t4_h,
             si_v, di_v, g1_v, g2_v, pn_v, t1_v, t2_v, t3_v, t4_v,
             sg1, sg2, wg):
    cid = lax.axis_index("c")
    sid = lax.axis_index("s")
    wid = sid * NC + cid
    base = wid * EPW
    NCHUNK = EPW // CH
    pltpu.sync_copy(pn_h, pn_v)

    pltpu.sync_copy(src_h.at[pl.ds(base, CH)], si_v.at[0])
    pltpu.sync_copy(dst_h.at[pl.ds(base, CH)], di_v.at[0])
    pltpu.async_copy(b1h_h.at[si_v.at[0]], g1_v.at[0], sg1.at[0])
    pltpu.async_copy(b2h_h.at[di_v.at[0]], g2_v.at[0], sg2.at[0])

    def pair(m, carry):
        for b in (0, 1):
            k = 2 * m + b
            nb = 1 - b

            @pl.when(k + 1 < NCHUNK)
            def _():
                noff = base + (k + 1) * CH
                pltpu.sync_copy(src_h.at[pl.ds(noff, CH)], si_v.at[nb])
                pltpu.sync_copy(dst_h.at[pl.ds(noff, CH)], di_v.at[nb])

                @pl.when(k >= 1)
                def _():
                    pltpu.make_async_copy(g1_v.at[nb],
                                          g_h.at[pl.ds(base + (k - 1) * CH,
                                                       CH)],
                                          wg.at[nb]).wait()

                pltpu.async_copy(b1h_h.at[si_v.at[nb]], g1_v.at[nb],
                                 sg1.at[nb])
                pltpu.async_copy(b2h_h.at[di_v.at[nb]], g2_v.at[nb],
                                 sg2.at[nb])

            @pl.when(k < NCHUNK)
            def _():
              def tt(t, c):
                sl = pl.ds(t * L, L)
                tsl = pl.ds(k * CH + t * L, L)
                s16 = si_v[b, sl] * 4
                d16 = di_v[b, sl] * 4
                p1s = plsc.load_gather(pn_v, [s16])
                p3s = plsc.load_gather(pn_v, [s16 + 1])
                q1s = plsc.load_gather(pn_v, [s16 + 2])
                q3s = plsc.load_gather(pn_v, [s16 + 3])
                p1d = plsc.load_gather(pn_v, [d16])
                p3d = plsc.load_gather(pn_v, [d16 + 1])
                q1d = plsc.load_gather(pn_v, [d16 + 2])
                q3d = plsc.load_gather(pn_v, [d16 + 3])
                t1_v[tsl] = p1s + p3d
                t2_v[tsl] = p1d + p3s
                t3_v[tsl] = q1s + q3d
                t4_v[tsl] = q1d + q3s
                return c

              lax.fori_loop(0, CH // L, tt, 0)

              pltpu.make_async_copy(b1h_h.at[si_v.at[b]], g1_v.at[b],
                                    sg1.at[b]).wait()
              pltpu.make_async_copy(b2h_h.at[di_v.at[b]], g2_v.at[b],
                                    sg2.at[b]).wait()

              def row(i, c):
                for j in range(D // L):
                    sl = pl.ds(j * L, L)
                    g1_v[b, i, sl] = g1_v[b, i, sl] + g2_v[b, i, sl]
                return c

              lax.fori_loop(0, CH, row, 0)
              pltpu.async_copy(g1_v.at[b], g_h.at[pl.ds(base + k * CH, CH)],
                               wg.at[b])
        return carry

    lax.fori_loop(0, (NCHUNK + 1) // 2, pair, 0)
    pltpu.make_async_copy(g1_v.at[0],
                          g_h.at[pl.ds(base + (NCHUNK - 2) * CH, CH)],
                          wg.at[0]).wait()
    pltpu.make_async_copy(g1_v.at[1],
                          g_h.at[pl.ds(base + (NCHUNK - 1) * CH, CH)],
                          wg.at[1]).wait()
    pltpu.sync_copy(t1_v, t1_h.at[pl.ds(base, EPW)])
    pltpu.sync_copy(t2_v, t2_h.at[pl.ds(base, EPW)])
    pltpu.sync_copy(t3_v, t3_h.at[pl.ds(base, EPW)])
    pltpu.sync_copy(t4_v, t4_h.at[pl.ds(base, EPW)])


def _lrelu(x):
    return jnp.where(x >= 0.0, x, 0.01 * x)


def _c2_body(src_h, dst_h, t1_h, t2_h, t3_h, t4_h, pe_h, qe_h,
             exa_h, exr_h, exae_h, exare_h, sp_h,
             si_v, di_v, tb1, tb2, tb3, tb4, peb, qeb, ea_v, er_v, eae_v,
             eare_v, a0_v, a1_v, a2_v, a3_v, red_v, tmp_v, stage_s):
    cid = lax.axis_index("c")
    sid = lax.axis_index("s")
    wid = sid * NC + cid
    base = wid * EPW
    zero16 = jnp.zeros((L,), jnp.float32)

    def zacc(k, c):
        sl = pl.ds(k * L, L)
        a0_v[sl] = zero16
        a1_v[sl] = zero16
        a2_v[sl] = zero16
        a3_v[sl] = zero16
        return c

    lax.fori_loop(0, NP // L, zacc, 0)

    def chunk(k, carry):
        off = base + k * CH2
        sl_h = pl.ds(off, CH2)
        pltpu.sync_copy(src_h.at[sl_h], si_v)
        pltpu.sync_copy(dst_h.at[sl_h], di_v)
        pltpu.sync_copy(t1_h.at[sl_h], tb1)
        pltpu.sync_copy(t2_h.at[sl_h], tb2)
        pltpu.sync_copy(t3_h.at[sl_h], tb3)
        pltpu.sync_copy(t4_h.at[sl_h], tb4)
        pltpu.sync_copy(pe_h.at[sl_h], peb)
        pltpu.sync_copy(qe_h.at[sl_h], qeb)

        def tt(t, c):
            sl = pl.ds(t * L, L)
            s16 = si_v[sl]
            d16 = di_v[sl]
            pe = peb[sl]
            qe = qeb[sl]
            ea = jnp.exp(_lrelu(tb1[sl] + pe))
            er = jnp.exp(_lrelu(tb2[sl] + pe))
            eae = jnp.exp(_lrelu(tb3[sl] + qe))
            eare = jnp.exp(_lrelu(tb4[sl] + qe))
            ea_v[sl] = ea
            er_v[sl] = er
            eae_v[sl] = eae
            eare_v[sl] = eare
            plsc.addupdate_scatter(a0_v, [d16], ea)
            plsc.addupdate_scatter(a1_v, [s16], er)
            plsc.addupdate_scatter(a2_v, [d16], eae)
            plsc.addupdate_scatter(a3_v, [s16], eare)
            return c

        lax.fori_loop(0, CH2 // L, tt, 0)
        pltpu.sync_copy(ea_v, exa_h.at[sl_h])
        pltpu.sync_copy(er_v, exr_h.at[sl_h])
        pltpu.sync_copy(eae_v, exae_h.at[sl_h])
        pltpu.sync_copy(eare_v, exare_h.at[sl_h])
        return carry

    lax.fori_loop(0, EPW // CH2, chunk, 0)

    soff = sid * STR
    accs = [a0_v, a1_v, a2_v, a3_v]
    for r in range(4):
        pltpu.sync_copy(accs[r], stage_s.at[pl.ds(sid * NP, NP)])
        plsc.subcore_barrier()

        def zred(k, c):
            red_v[pl.ds(k * L, L)] = zero16
            return c

        lax.fori_loop(0, STR // L, zred, 0)

        def rt(t, c):
            pltpu.sync_copy(stage_s.at[pl.ds(t * NP + soff, STR)], tmp_v)

            def add(k, cc):
                sl = pl.ds(k * L, L)
                red_v[sl] = red_v[sl] + tmp_v[sl]
                return cc

            lax.fori_loop(0, STR // L, add, 0)
            return c

        lax.fori_loop(0, NS, rt, 0)
        pltpu.sync_copy(red_v, sp_h.at[pl.ds((cid * 4 + r) * NP + soff, STR)])
        plsc.subcore_barrier()


def _d_body(src_h, dst_h, z_h, ej_h, exa_h, exr_h, exae_h, exare_h, sp_h,
            h1_h, h2_h,
            si_v, di_v, r1_v, r2_v, ea_v, er_v, al_v, ar_v,
            sA_v, sB_v, tmp_v, zb_v, acc_s, sld, sg1, sg2, ssc):
    cid = lax.axis_index("c")
    sid = lax.axis_index("s")
    rA = 2 * cid
    NCHUNK = EPT // CHD

    pltpu.sync_copy(sp_h.at[pl.ds(rA * NP, NP)], sA_v)
    pltpu.sync_copy(sp_h.at[pl.ds((rA + 1) * NP, NP)], sB_v)

    def addpt(pt, c):
        pltpu.sync_copy(sp_h.at[pl.ds((4 + rA) * NP + pt * STR, STR)], tmp_v)

        def addA(k, cc):
            sl = pl.ds(pt * STR + k * L, L)
            sA_v[sl] = sA_v[sl] + tmp_v[pl.ds(k * L, L)]
            return cc

        lax.fori_loop(0, STR // L, addA, 0)
        pltpu.sync_copy(sp_h.at[pl.ds((4 + rA + 1) * NP + pt * STR, STR)],
                        tmp_v)

        def addB(k, cc):
            sl = pl.ds(pt * STR + k * L, L)
            sB_v[sl] = sB_v[sl] + tmp_v[pl.ds(k * L, L)]
            return cc

        lax.fori_loop(0, STR // L, addB, 0)
        return c

    lax.fori_loop(0, NS, addpt, 0)

    zero16 = jnp.zeros((L,), jnp.float32)

    def zrow(i, c):
        for j in range(D // L):
            zb_v[i, pl.ds(j * L, L)] = zero16
        return c

    lax.fori_loop(0, 8, zrow, 0)

    def zcp(rep, c):
        pltpu.sync_copy(zb_v, acc_s.at[pl.ds(sid * STR + rep * 8, 8)])
        return c

    lax.fori_loop(0, STR // 8, zcp, 0)
    plsc.subcore_barrier()

    base = sid * EPT

    def issue_loads(k, b):
        off = base + k * CHD
        sl_h = pl.ds(off, CHD)
        pltpu.async_copy(src_h.at[sl_h], si_v.at[b], sld.at[b])
        pltpu.async_copy(dst_h.at[sl_h], di_v.at[b], sld.at[b])

        @pl.when(cid == 0)
        def _():
            pltpu.async_copy(exa_h.at[sl_h], ea_v.at[b], sld.at[b])
            pltpu.async_copy(exr_h.at[sl_h], er_v.at[b], sld.at[b])

        @pl.when(cid == 1)
        def _():
            pltpu.async_copy(exae_h.at[sl_h], ea_v.at[b], sld.at[b])
            pltpu.async_copy(exare_h.at[sl_h], er_v.at[b], sld.at[b])

    def issue_rows(k, b):
        off = base + k * CHD

        @pl.when(cid == 0)
        def _():
            pltpu.async_copy(z_h.at[si_v.at[b]], r1_v.at[b], sg1.at[b])
            pltpu.async_copy(z_h.at[di_v.at[b]], r2_v.at[b], sg2.at[b])

        @pl.when(cid == 1)
        def _():
            pltpu.async_copy(ej_h.at[pl.ds(off, CHD)], r1_v.at[b], sg1.at[b])

    def wait_loads(k, b):
        off = base + k * CHD
        sl_h = pl.ds(off, CHD)
        pltpu.make_async_copy(src_h.at[sl_h], si_v.at[b], sld.at[b]).wait()
        pltpu.make_async_copy(dst_h.at[sl_h], di_v.at[b], sld.at[b]).wait()
        pltpu.make_async_copy(exa_h.at[sl_h], ea_v.at[b], sld.at[b]).wait()
        pltpu.make_async_copy(exr_h.at[sl_h], er_v.at[b], sld.at[b]).wait()


    issue_loads(0, 0)
    pltpu.make_async_copy(src_h.at[pl.ds(base, CHD)], si_v.at[0],
                          sld.at[0]).wait()
    pltpu.make_async_copy(dst_h.at[pl.ds(base, CHD)], di_v.at[0],
                          sld.at[0]).wait()
    pltpu.make_async_copy(exa_h.at[pl.ds(base, CHD)], ea_v.at[0],
                          sld.at[0]).wait()
    pltpu.make_async_copy(exr_h.at[pl.ds(base, CHD)], er_v.at[0],
                          sld.at[0]).wait()
    issue_rows(0, 0)

    def pair(m, carry):
        for b in (0, 1):
            k = 2 * m + b
            nb = 1 - b

            @pl.when(k + 1 < NCHUNK)
            def _():
                issue_loads(k + 1, nb)

            @pl.when(k < NCHUNK)
            def _():
                @pl.when(cid == 0)
                def _():
                    pltpu.make_async_copy(z_h.at[si_v.at[b]], r1_v.at[b],
                                          sg1.at[b]).wait()
                    pltpu.make_async_copy(z_h.at[di_v.at[b]], r2_v.at[b],
                                          sg2.at[b]).wait()

                @pl.when(cid == 1)
                def _():
                    pltpu.make_async_copy(
                        ej_h.at[pl.ds(base + k * CHD, CHD)], r1_v.at[b],
                        sg1.at[b]).wait()

                def tt(t, c):
                    sl = pl.ds(t * L, L)
                    s16 = si_v[b, sl]
                    d16 = di_v[b, sl]
                    den = plsc.load_gather(sA_v, [d16])
                    al_v[sl] = ea_v[b, sl] / (den + 1e-9)
                    den2 = plsc.load_gather(sB_v, [s16])
                    ar_v[sl] = er_v[b, sl] / (den2 + 1e-9)
                    return c

                lax.fori_loop(0, CHD // L, tt, 0)

                @pl.when(cid == 0)
                def _():
                    def grp(t, c):
                        al16 = al_v[pl.ds(t * L, L)]
                        ar16 = ar_v[pl.ds(t * L, L)]
                        for lane in range(L):
                            i = t * L + lane
                            a = al16[lane]
                            bb = ar16[lane]
                            for j in range(D // L):
                                sl = pl.ds(j * L, L)
                                r1_v[b, i, sl] = r1_v[b, i, sl] * a
                                r2_v[b, i, sl] = r2_v[b, i, sl] * bb
                        return c

                    lax.fori_loop(0, CHD // L, grp, 0)

                @pl.when(cid == 1)
                def _():
                    def grp(t, c):
                        al16 = al_v[pl.ds(t * L, L)]
                        ar16 = ar_v[pl.ds(t * L, L)]
                        for lane in range(L):
                            i = t * L + lane
                            a = al16[lane]
                            bb = ar16[lane]
                            for j in range(D // L):
                                sl = pl.ds(j * L, L)
                                x = r1_v[b, i, sl]
                                r2_v[b, i, sl] = x * bb
                                r1_v[b, i, sl] = x * a
                        return c

                    lax.fori_loop(0, CHD // L, grp, 0)

                pltpu.sync_copy(r1_v.at[b], acc_s.at[di_v.at[b]], add=True)
                pltpu.sync_copy(r2_v.at[b], acc_s.at[si_v.at[b]], add=True)

                @pl.when(k + 1 < NCHUNK)
                def _():
                    wait_loads(k + 1, nb)
                    issue_rows(k + 1, nb)
        return carry

    lax.fori_loop(0, (NCHUNK + 1) // 2, pair, 0)
    plsc.subcore_barrier()
    row_sl = pl.ds(sid * STR, STR)

    @pl.when(cid == 0)
    def _():
        pltpu.sync_copy(acc_s.at[row_sl], h1_h.at[row_sl])

    @pl.when(cid == 1)
    def _():
        pltpu.sync_copy(acc_s.at[row_sl], h2_h.at[row_sl])


def kernel(h, e, edge_index, W_fc, W_fce, W_attn, W_attn_e, W_B1, b_B1, W_B2,
           b_B2, W_B3, b_B3, W_mix, b_mix, ln_gamma, ln_beta):
    src = edge_index[0]
    dst = edge_index[1]

    w1, w2, w3 = W_attn[0, :D], W_attn[0, D:2 * D], W_attn[0, 2 * D:]
    w1e, w2e, w3e = W_attn_e[0, :D], W_attn_e[0, D:2 * D], W_attn_e[0, 2 * D:]
    u2 = (W_fce.T @ w2)[:, None]
    u2e = (W_fce.T @ w2e)[:, None]
    Wp = jnp.stack([w1, w3, w1e, w3e], axis=1)
    Wm1 = W_mix[:, :D]
    Wm2 = W_mix[:, D:]
    WcT = (Wm2 @ W_fce).T

    f32 = jnp.float32
    BN = 1000
    rep = lambda: pl.BlockSpec((D, D), lambda i: (0, 0))
    vec = lambda: pl.BlockSpec((D,), lambda i: (0,))

    z, b1h, b2h, pn = pl.pallas_call(
        _node_body,
        grid=(N // BN,),
        in_specs=[pl.BlockSpec((BN, D), lambda i: (i, 0)),
                  rep(), rep(), vec(), rep(), vec(),
                  pl.BlockSpec((D, 4), lambda i: (0, 0))],
        out_specs=[pl.BlockSpec((BN, D), lambda i: (i, 0))] * 3 + [
            pl.BlockSpec((BN, 4), lambda i: (i, 0))],
        out_shape=[jax.ShapeDtypeStruct((N, D), f32)] * 3 + [
            jax.ShapeDtypeStruct((N, 4), f32)],
    )(h, W_fc.T, W_B1.T, b_B1, W_B2.T, b_B2, Wp)

    mesh = plsc.VectorSubcoreMesh(core_axis_name="c", subcore_axis_name="s")
    c1 = pl.kernel(
        _c1_body,
        out_type=[jax.ShapeDtypeStruct((E, D), f32)] + [
            jax.ShapeDtypeStruct((E,), f32)] * 4,
        mesh=mesh,
        compiler_params=pltpu.CompilerParams(needs_layout_passes=False),
        scratch_types=[
            pltpu.VMEM((2, CH), jnp.int32), pltpu.VMEM((2, CH), jnp.int32),
            pltpu.VMEM((2, CH, D), f32), pltpu.VMEM((2, CH, D), f32),
            pltpu.VMEM((N * 4,), f32),
            pltpu.VMEM((EPW,), f32), pltpu.VMEM((EPW,), f32),
            pltpu.VMEM((EPW,), f32), pltpu.VMEM((EPW,), f32),
            pltpu.SemaphoreType.DMA((2,)), pltpu.SemaphoreType.DMA((2,)),
            pltpu.SemaphoreType.DMA((2,)),
        ],
    )
    g, t1, t2, t3, t4 = c1(src, dst, b1h, b2h, pn.reshape(N * 4))

    BE = 2000
    e_ji, pe, qe = pl.pallas_call(
        _edge_body,
        grid=(E // BE,),
        in_specs=[pl.BlockSpec((BE, D), lambda i: (i, 0)),
                  pl.BlockSpec((BE, D), lambda i: (i, 0)),
                  rep(), vec(),
                  pl.BlockSpec((D, 1), lambda i: (0, 0)),
                  pl.BlockSpec((D, 1), lambda i: (0, 0))],
        out_specs=[pl.BlockSpec((BE, D), lambda i: (i, 0)),
                   pl.BlockSpec((BE, 1), lambda i: (i, 0)),
                   pl.BlockSpec((BE, 1), lambda i: (i, 0))],
        out_shape=[jax.ShapeDtypeStruct((E, D), f32),
                   jax.ShapeDtypeStruct((E, 1), f32),
                   jax.ShapeDtypeStruct((E, 1), f32)],
    )(e, g, W_B3.T, b_B3, u2, u2e)
    pe1 = pe.reshape(E)
    qe1 = qe.reshape(E)

    c2 = pl.kernel(
        _c2_body,
        out_type=[jax.ShapeDtypeStruct((E,), f32)] * 4 + [
            jax.ShapeDtypeStruct((NC * 4 * NP,), f32)],
        mesh=mesh,
        compiler_params=pltpu.CompilerParams(needs_layout_passes=False),
        scratch_types=[
            pltpu.VMEM((CH2,), jnp.int32), pltpu.VMEM((CH2,), jnp.int32)] +
            [pltpu.VMEM((CH2,), f32)] * 10 +
            [pltpu.VMEM((NP,), f32)] * 4 +
            [pltpu.VMEM((STR,), f32), pltpu.VMEM((STR,), f32),
             pltpu.VMEM_SHARED((NS * NP,), f32)],
    )
    exa, exr, exae, exare, sp = c2(src, dst, t1, t2, t3, t4, pe1, qe1)

    dk = pl.kernel(
        _d_body,
        out_type=[jax.ShapeDtypeStruct((NP, D), f32),
                  jax.ShapeDtypeStruct((NP, D), f32)],
        mesh=mesh,
        compiler_params=pltpu.CompilerParams(needs_layout_passes=False),
        scratch_types=[
            pltpu.VMEM((2, CHD), jnp.int32), pltpu.VMEM((2, CHD), jnp.int32),
            pltpu.VMEM((2, CHD, D), f32), pltpu.VMEM((2, CHD, D), f32),
            pltpu.VMEM((2, CHD), f32), pltpu.VMEM((2, CHD), f32),
            pltpu.VMEM((CHD,), f32), pltpu.VMEM((CHD,), f32),
            pltpu.VMEM((NP,), f32), pltpu.VMEM((NP,), f32),
            pltpu.VMEM((STR,), f32),
            pltpu.VMEM((8, D), f32),
            pltpu.VMEM_SHARED((NP, D), f32),
            pltpu.SemaphoreType.DMA((2,)), pltpu.SemaphoreType.DMA((2,)),
            pltpu.SemaphoreType.DMA((2,)), pltpu.SemaphoreType.DMA((2,)),
        ],
    )
    h1p, h2pp = dk(src, dst, z, e_ji, exa, exr, exae, exare, sp)
    h1 = h1p[:N]
    h2p = h2pp[:N]

    h_out = pl.pallas_call(
        _tail_body,
        grid=(N // BN,),
        in_specs=[pl.BlockSpec((BN, D), lambda i: (i, 0))] * 3 + [
            rep(), rep(), vec(), vec(), vec()],
        out_specs=pl.BlockSpec((BN, D), lambda i: (i, 0)),
        out_shape=jax.ShapeDtypeStruct((N, D), f32),
    )(h1, h2p, h, Wm1.T, WcT, b_mix, ln_gamma, ln_beta)

    return (h_out, e_ji)


# speedup vs baseline: 1.1110x; 1.1110x over previous
import jax
import jax.numpy as jnp
from jax import lax
from jax.experimental import pallas as pl
from jax.experimental.pallas import tpu as pltpu
from jax.experimental.pallas import tpu_sc as plsc

N = 10000
E = 320000
D = 128
NC = 2
NS = 16
NW = NC * NS
L = 16
NP = 10240
EPW = E // NW
EPT = E // NS
CH = 80
CH2 = 2000
CHD = 32
RPS = N // NS
STR = NP // NS


def _node_body(h_ref, wfct, wb1t, b1, wb2t, b2, wp, z_ref, b1h_ref, b2h_ref,
               pn_ref):
    hb = h_ref[...]
    z = jnp.dot(hb, wfct[...], preferred_element_type=jnp.float32)
    z_ref[...] = z
    b1h_ref[...] = jnp.dot(hb, wb1t[...],
                           preferred_element_type=jnp.float32) + b1[...][None, :]
    b2h_ref[...] = jnp.dot(hb, wb2t[...],
                           preferred_element_type=jnp.float32) + b2[...][None, :]
    pn_ref[...] = jnp.dot(z, wp[...], preferred_element_type=jnp.float32)


def _edge_body(e_ref, g_ref, wb3t, b3, u2, u2e, ej_ref, pe_ref, qe_ref):
    eb = e_ref[...]
    ej = jax.nn.relu(g_ref[...]
                     + jnp.dot(eb, wb3t[...], preferred_element_type=jnp.float32)
                     + b3[...][None, :]) + eb
    ej_ref[...] = ej
    pe_ref[...] = jnp.dot(ej, u2[...], preferred_element_type=jnp.float32)
    qe_ref[...] = jnp.dot(ej, u2e[...], preferred_element_type=jnp.float32)


def _tail_body(h1_ref, h2_ref, h_ref, wm1t, wct, bmix, gamma, beta, out_ref):
    hs = (jnp.dot(h1_ref[...], wm1t[...], preferred_element_type=jnp.float32)
          + jnp.dot(h2_ref[...], wct[...], preferred_element_type=jnp.float32)
          + 2.0 * bmix[...][None, :])
    mu = jnp.mean(hs, axis=-1, keepdims=True)
    var = jnp.mean((hs - mu) ** 2, axis=-1, keepdims=True)
    hn = (hs - mu) / jnp.sqrt(var + 1e-5) * gamma[...][None, :] + beta[...][None, :]
    out_ref[...] = jax.nn.relu(hn) + h_ref[...]


def _c1_body(src_h, dst_h, b1h_h, b2h_h, pn_h, g_h, t1_h, t2_h, t3_h, t4_h,
             si_v, di_v, g1_v, g2_v, pn_v, t1_v, t2_v, t3_v, t4_v,
             sg1, sg2, wg):
    cid = lax.axis_index("c")
    sid = lax.axis_index("s")
    wid = sid * NC + cid
    base = wid * EPW
    NCHUNK = EPW // CH
    pltpu.sync_copy(pn_h, pn_v)

    pltpu.sync_copy(src_h.at[pl.ds(base, CH)], si_v.at[0])
    pltpu.sync_copy(dst_h.at[pl.ds(base, CH)], di_v.at[0])
    pltpu.async_copy(b1h_h.at[si_v.at[0]], g1_v.at[0], sg1.at[0])
    pltpu.async_copy(b2h_h.at[di_v.at[0]], g2_v.at[0], sg2.at[0])

    def pair(m, carry):
        for b in (0, 1):
            k = 2 * m + b
            nb = 1 - b

            @pl.when(k + 1 < NCHUNK)
            def _():
                noff = base + (k + 1) * CH
                pltpu.sync_copy(src_h.at[pl.ds(noff, CH)], si_v.at[nb])
                pltpu.sync_copy(dst_h.at[pl.ds(noff, CH)], di_v.at[nb])

                @pl.when(k >= 1)
                def _():
                    pltpu.make_async_copy(g1_v.at[nb],
                                          g_h.at[pl.ds(base + (k - 1) * CH,
                                                       CH)],
                                          wg.at[nb]).wait()

                pltpu.async_copy(b1h_h.at[si_v.at[nb]], g1_v.at[nb],
                                 sg1.at[nb])
                pltpu.async_copy(b2h_h.at[di_v.at[nb]], g2_v.at[nb],
                                 sg2.at[nb])

            @pl.when(k < NCHUNK)
            def _():
              def tt(t, c):
                sl = pl.ds(t * L, L)
                tsl = pl.ds(k * CH + t * L, L)
                s16 = si_v[b, sl] * 4
                d16 = di_v[b, sl] * 4
                p1s = plsc.load_gather(pn_v, [s16])
                p3s = plsc.load_gather(pn_v, [s16 + 1])
                q1s = plsc.load_gather(pn_v, [s16 + 2])
                q3s = plsc.load_gather(pn_v, [s16 + 3])
                p1d = plsc.load_gather(pn_v, [d16])
                p3d = plsc.load_gather(pn_v, [d16 + 1])
                q1d = plsc.load_gather(pn_v, [d16 + 2])
                q3d = plsc.load_gather(pn_v, [d16 + 3])
                t1_v[tsl] = p1s + p3d
                t2_v[tsl] = p1d + p3s
                t3_v[tsl] = q1s + q3d
                t4_v[tsl] = q1d + q3s
                return c

              lax.fori_loop(0, CH // L, tt, 0)

              pltpu.make_async_copy(b1h_h.at[si_v.at[b]], g1_v.at[b],
                                    sg1.at[b]).wait()
              pltpu.make_async_copy(b2h_h.at[di_v.at[b]], g2_v.at[b],
                                    sg2.at[b]).wait()

              def row(i, c):
                for j in range(D // L):
                    sl = pl.ds(j * L, L)
                    g1_v[b, i, sl] = g1_v[b, i, sl] + g2_v[b, i, sl]
                return c

              lax.fori_loop(0, CH, row, 0)
              pltpu.async_copy(g1_v.at[b], g_h.at[pl.ds(base + k * CH, CH)],
                               wg.at[b])
        return carry

    lax.fori_loop(0, (NCHUNK + 1) // 2, pair, 0)
    pltpu.make_async_copy(g1_v.at[0],
                          g_h.at[pl.ds(base + (NCHUNK - 2) * CH, CH)],
                          wg.at[0]).wait()
    pltpu.make_async_copy(g1_v.at[1],
                          g_h.at[pl.ds(base + (NCHUNK - 1) * CH, CH)],
                          wg.at[1]).wait()
    pltpu.sync_copy(t1_v, t1_h.at[pl.ds(base, EPW)])
    pltpu.sync_copy(t2_v, t2_h.at[pl.ds(base, EPW)])
    pltpu.sync_copy(t3_v, t3_h.at[pl.ds(base, EPW)])
    pltpu.sync_copy(t4_v, t4_h.at[pl.ds(base, EPW)])


def _lrelu(x):
    return jnp.where(x >= 0.0, x, 0.01 * x)


def _c2_body(src_h, dst_h, t1_h, t2_h, t3_h, t4_h, pe_h, qe_h,
             exa_h, exr_h, exae_h, exare_h, sp_h,
             si_v, di_v, tb1, tb2, tb3, tb4, peb, qeb, ea_v, er_v, eae_v,
             eare_v, a0_v, a1_v, a2_v, a3_v, red_v, tmp_v, stage_s):
    cid = lax.axis_index("c")
    sid = lax.axis_index("s")
    wid = sid * NC + cid
    base = wid * EPW
    zero16 = jnp.zeros((L,), jnp.float32)

    def zacc(k, c):
        sl = pl.ds(k * L, L)
        a0_v[sl] = zero16
        a1_v[sl] = zero16
        a2_v[sl] = zero16
        a3_v[sl] = zero16
        return c

    lax.fori_loop(0, NP // L, zacc, 0)

    def chunk(k, carry):
        off = base + k * CH2
        sl_h = pl.ds(off, CH2)
        pltpu.sync_copy(src_h.at[sl_h], si_v)
        pltpu.sync_copy(dst_h.at[sl_h], di_v)
        pltpu.sync_copy(t1_h.at[sl_h], tb1)
        pltpu.sync_copy(t2_h.at[sl_h], tb2)
        pltpu.sync_copy(t3_h.at[sl_h], tb3)
        pltpu.sync_copy(t4_h.at[sl_h], tb4)
        pltpu.sync_copy(pe_h.at[sl_h], peb)
        pltpu.sync_copy(qe_h.at[sl_h], qeb)

        def tt(t, c):
            sl = pl.ds(t * L, L)
            s16 = si_v[sl]
            d16 = di_v[sl]
            pe = peb[sl]
            qe = qeb[sl]
            ea = jnp.exp(_lrelu(tb1[sl] + pe))
            er = jnp.exp(_lrelu(tb2[sl] + pe))
            eae = jnp.exp(_lrelu(tb3[sl] + qe))
            eare = jnp.exp(_lrelu(tb4[sl] + qe))
            ea_v[sl] = ea
            er_v[sl] = er
            eae_v[sl] = eae
            eare_v[sl] = eare
            plsc.addupdate_scatter(a0_v, [d16], ea)
            plsc.addupdate_scatter(a1_v, [s16], er)
            plsc.addupdate_scatter(a2_v, [d16], eae)
            plsc.addupdate_scatter(a3_v, [s16], eare)
            return c

        lax.fori_loop(0, CH2 // L, tt, 0)
        pltpu.sync_copy(ea_v, exa_h.at[sl_h])
        pltpu.sync_copy(er_v, exr_h.at[sl_h])
        pltpu.sync_copy(eae_v, exae_h.at[sl_h])
        pltpu.sync_copy(eare_v, exare_h.at[sl_h])
        return carry

    lax.fori_loop(0, EPW // CH2, chunk, 0)

    soff = sid * STR
    accs = [a0_v, a1_v, a2_v, a3_v]
    for r in range(4):
        pltpu.sync_copy(accs[r], stage_s.at[pl.ds(sid * NP, NP)])
        plsc.subcore_barrier()

        def zred(k, c):
            red_v[pl.ds(k * L, L)] = zero16
            return c

        lax.fori_loop(0, STR // L, zred, 0)

        def rt(t, c):
            pltpu.sync_copy(stage_s.at[pl.ds(t * NP + soff, STR)], tmp_v)

            def add(k, cc):
                sl = pl.ds(k * L, L)
                red_v[sl] = red_v[sl] + tmp_v[sl]
                return cc

            lax.fori_loop(0, STR // L, add, 0)
            return c

        lax.fori_loop(0, NS, rt, 0)
        pltpu.sync_copy(red_v, sp_h.at[pl.ds((cid * 4 + r) * NP + soff, STR)])
        plsc.subcore_barrier()


def _d_body(src_h, dst_h, z_h, ej_h, exa_h, exr_h, exae_h, exare_h, sp_h,
            h1_h, h2_h,
            si_v, di_v, r1_v, r2_v, ea_v, er_v, al_v, ar_v,
            sA_v, sB_v, tmp_v, zb_v, acc_s, sem1, sem2):
    cid = lax.axis_index("c")
    sid = lax.axis_index("s")
    rA = 2 * cid

    pltpu.sync_copy(sp_h.at[pl.ds(rA * NP, NP)], sA_v)
    pltpu.sync_copy(sp_h.at[pl.ds((rA + 1) * NP, NP)], sB_v)

    def addpt(pt, c):
        pltpu.sync_copy(sp_h.at[pl.ds((4 + rA) * NP + pt * STR, STR)], tmp_v)

        def addA(k, cc):
            sl = pl.ds(pt * STR + k * L, L)
            sA_v[sl] = sA_v[sl] + tmp_v[pl.ds(k * L, L)]
            return cc

        lax.fori_loop(0, STR // L, addA, 0)
        pltpu.sync_copy(sp_h.at[pl.ds((4 + rA + 1) * NP + pt * STR, STR)],
                        tmp_v)

        def addB(k, cc):
            sl = pl.ds(pt * STR + k * L, L)
            sB_v[sl] = sB_v[sl] + tmp_v[pl.ds(k * L, L)]
            return cc

        lax.fori_loop(0, STR // L, addB, 0)
        return c

    lax.fori_loop(0, NS, addpt, 0)

    zero16 = jnp.zeros((L,), jnp.float32)

    def zrow(i, c):
        for j in range(D // L):
            zb_v[i, pl.ds(j * L, L)] = zero16
        return c

    lax.fori_loop(0, 8, zrow, 0)

    def zcp(rep, c):
        pltpu.sync_copy(zb_v, acc_s.at[pl.ds(sid * STR + rep * 8, 8)])
        return c

    lax.fori_loop(0, STR // 8, zcp, 0)
    plsc.subcore_barrier()

    base = sid * EPT

    def chunk(k, carry):
        off = base + k * CH
        sl_h = pl.ds(off, CH)
        pltpu.sync_copy(src_h.at[sl_h], si_v)
        pltpu.sync_copy(dst_h.at[sl_h], di_v)

        @pl.when(cid == 0)
        def _():
            d1 = pltpu.async_copy(z_h.at[si_v], r1_v, sem1)
            d2 = pltpu.async_copy(z_h.at[di_v], r2_v, sem2)
            pltpu.sync_copy(exa_h.at[sl_h], ea_v)
            pltpu.sync_copy(exr_h.at[sl_h], er_v)
            d1.wait()
            d2.wait()

        @pl.when(cid == 1)
        def _():
            pltpu.sync_copy(ej_h.at[sl_h], r1_v)
            pltpu.sync_copy(exae_h.at[sl_h], ea_v)
            pltpu.sync_copy(exare_h.at[sl_h], er_v)

        def tt(t, c):
            sl = pl.ds(t * L, L)
            s16 = si_v[sl]
            d16 = di_v[sl]
            den = plsc.load_gather(sA_v, [d16])
            al_v[sl] = ea_v[sl] / (den + 1e-9)
            den2 = plsc.load_gather(sB_v, [s16])
            ar_v[sl] = er_v[sl] / (den2 + 1e-9)
            return c

        lax.fori_loop(0, CH // L, tt, 0)

        @pl.when(cid == 0)
        def _():
            def grp(t, c):
                al16 = al_v[pl.ds(t * L, L)]
                ar16 = ar_v[pl.ds(t * L, L)]
                for lane in range(L):
                    i = t * L + lane
                    a = al16[lane]
                    b = ar16[lane]
                    for j in range(D // L):
                        sl = pl.ds(j * L, L)
                        r1_v[i, sl] = r1_v[i, sl] * a
                        r2_v[i, sl] = r2_v[i, sl] * b
                return c

            lax.fori_loop(0, CH // L, grp, 0)

        @pl.when(cid == 1)
        def _():
            def grp(t, c):
                al16 = al_v[pl.ds(t * L, L)]
                ar16 = ar_v[pl.ds(t * L, L)]
                for lane in range(L):
                    i = t * L + lane
                    a = al16[lane]
                    b = ar16[lane]
                    for j in range(D // L):
                        sl = pl.ds(j * L, L)
                        x = r1_v[i, sl]
                        r2_v[i, sl] = x * b
                        r1_v[i, sl] = x * a
                return c

            lax.fori_loop(0, CH // L, grp, 0)

        pltpu.sync_copy(r1_v, acc_s.at[di_v], add=True)
        pltpu.sync_copy(r2_v, acc_s.at[si_v], add=True)
        return carry

    lax.fori_loop(0, EPT // CH, chunk, 0)
    plsc.subcore_barrier()
    row_sl = pl.ds(sid * STR, STR)

    @pl.when(cid == 0)
    def _():
        pltpu.sync_copy(acc_s.at[row_sl], h1_h.at[row_sl])

    @pl.when(cid == 1)
    def _():
        pltpu.sync_copy(acc_s.at[row_sl], h2_h.at[row_sl])


def kernel(h, e, edge_index, W_fc, W_fce, W_attn, W_attn_e, W_B1, b_B1, W_B2,
           b_B2, W_B3, b_B3, W_mix, b_mix, ln_gamma, ln_beta):
    src = edge_index[0]
    dst = edge_index[1]

    w1, w2, w3 = W_attn[0, :D], W_attn[0, D:2 * D], W_attn[0, 2 * D:]
    w1e, w2e, w3e = W_attn_e[0, :D], W_attn_e[0, D:2 * D], W_attn_e[0, 2 * D:]
    u2 = (W_fce.T @ w2)[:, None]
    u2e = (W_fce.T @ w2e)[:, None]
    Wp = jnp.stack([w1, w3, w1e, w3e], axis=1)
    Wm1 = W_mix[:, :D]
    Wm2 = W_mix[:, D:]
    WcT = (Wm2 @ W_fce).T

    f32 = jnp.float32
    BN = 1000
    rep = lambda: pl.BlockSpec((D, D), lambda i: (0, 0))
    vec = lambda: pl.BlockSpec((D,), lambda i: (0,))

    z, b1h, b2h, pn = pl.pallas_call(
        _node_body,
        grid=(N // BN,),
        in_specs=[pl.BlockSpec((BN, D), lambda i: (i, 0)),
                  rep(), rep(), vec(), rep(), vec(),
                  pl.BlockSpec((D, 4), lambda i: (0, 0))],
        out_specs=[pl.BlockSpec((BN, D), lambda i: (i, 0))] * 3 + [
            pl.BlockSpec((BN, 4), lambda i: (i, 0))],
        out_shape=[jax.ShapeDtypeStruct((N, D), f32)] * 3 + [
            jax.ShapeDtypeStruct((N, 4), f32)],
    )(h, W_fc.T, W_B1.T, b_B1, W_B2.T, b_B2, Wp)

    mesh = plsc.VectorSubcoreMesh(core_axis_name="c", subcore_axis_name="s")
    c1 = pl.kernel(
        _c1_body,
        out_type=[jax.ShapeDtypeStruct((E, D), f32)] + [
            jax.ShapeDtypeStruct((E,), f32)] * 4,
        mesh=mesh,
        compiler_params=pltpu.CompilerParams(needs_layout_passes=False),
        scratch_types=[
            pltpu.VMEM((2, CH), jnp.int32), pltpu.VMEM((2, CH), jnp.int32),
            pltpu.VMEM((2, CH, D), f32), pltpu.VMEM((2, CH, D), f32),
            pltpu.VMEM((N * 4,), f32),
            pltpu.VMEM((EPW,), f32), pltpu.VMEM((EPW,), f32),
            pltpu.VMEM((EPW,), f32), pltpu.VMEM((EPW,), f32),
            pltpu.SemaphoreType.DMA((2,)), pltpu.SemaphoreType.DMA((2,)),
            pltpu.SemaphoreType.DMA((2,)),
        ],
    )
    g, t1, t2, t3, t4 = c1(src, dst, b1h, b2h, pn.reshape(N * 4))

    BE = 2000
    e_ji, pe, qe = pl.pallas_call(
        _edge_body,
        grid=(E // BE,),
        in_specs=[pl.BlockSpec((BE, D), lambda i: (i, 0)),
                  pl.BlockSpec((BE, D), lambda i: (i, 0)),
                  rep(), vec(),
                  pl.BlockSpec((D, 1), lambda i: (0, 0)),
                  pl.BlockSpec((D, 1), lambda i: (0, 0))],
        out_specs=[pl.BlockSpec((BE, D), lambda i: (i, 0)),
                   pl.BlockSpec((BE, 1), lambda i: (i, 0)),
                   pl.BlockSpec((BE, 1), lambda i: (i, 0))],
        out_shape=[jax.ShapeDtypeStruct((E, D), f32),
                   jax.ShapeDtypeStruct((E, 1), f32),
                   jax.ShapeDtypeStruct((E, 1), f32)],
    )(e, g, W_B3.T, b_B3, u2, u2e)
    pe1 = pe.reshape(E)
    qe1 = qe.reshape(E)

    c2 = pl.kernel(
        _c2_body,
        out_type=[jax.ShapeDtypeStruct((E,), f32)] * 4 + [
            jax.ShapeDtypeStruct((NC * 4 * NP,), f32)],
        mesh=mesh,
        compiler_params=pltpu.CompilerParams(needs_layout_passes=False),
        scratch_types=[
            pltpu.VMEM((CH2,), jnp.int32), pltpu.VMEM((CH2,), jnp.int32)] +
            [pltpu.VMEM((CH2,), f32)] * 10 +
            [pltpu.VMEM((NP,), f32)] * 4 +
            [pltpu.VMEM((STR,), f32), pltpu.VMEM((STR,), f32),
             pltpu.VMEM_SHARED((NS * NP,), f32)],
    )
    exa, exr, exae, exare, sp = c2(src, dst, t1, t2, t3, t4, pe1, qe1)

    dk = pl.kernel(
        _d_body,
        out_type=[jax.ShapeDtypeStruct((NP, D), f32),
                  jax.ShapeDtypeStruct((NP, D), f32)],
        mesh=mesh,
        compiler_params=pltpu.CompilerParams(needs_layout_passes=False),
        scratch_types=[
            pltpu.VMEM((CH,), jnp.int32), pltpu.VMEM((CH,), jnp.int32),
            pltpu.VMEM((CH, D), f32), pltpu.VMEM((CH, D), f32),
            pltpu.VMEM((CH,), f32), pltpu.VMEM((CH,), f32),
            pltpu.VMEM((CH,), f32), pltpu.VMEM((CH,), f32),
            pltpu.VMEM((NP,), f32), pltpu.VMEM((NP,), f32),
            pltpu.VMEM((STR,), f32),
            pltpu.VMEM((8, D), f32),
            pltpu.VMEM_SHARED((NP, D), f32),
            pltpu.SemaphoreType.DMA, pltpu.SemaphoreType.DMA,
        ],
    )
    h1p, h2pp = dk(src, dst, z, e_ji, exa, exr, exae, exare, sp)
    h1 = h1p[:N]
    h2p = h2pp[:N]

    h_out = pl.pallas_call(
        _tail_body,
        grid=(N // BN,),
        in_specs=[pl.BlockSpec((BN, D), lambda i: (i, 0))] * 3 + [
            rep(), rep(), vec(), vec(), vec()],
        out_specs=pl.BlockSpec((BN, D), lambda i: (i, 0)),
        out_shape=jax.ShapeDtypeStruct((N, D), f32),
    )(h1, h2p, h, Wm1.T, WcT, b_mix, ln_gamma, ln_beta)

    return (h_out, e_ji)


# speedup vs baseline: 1.3250x; 1.1926x over previous
import jax
import jax.numpy as jnp
from jax import lax
from jax.experimental import pallas as pl
from jax.experimental.pallas import tpu as pltpu
from jax.experimental.pallas import tpu_sc as plsc

N = 10000
E = 320000
D = 128
NC = 2
NS = 16
NW = NC * NS
L = 16
NP = 10240
EPW = E // NW
EPT = E // NS
CH = 80
CH2 = 2000
CHD = 32
RPS = N // NS
STR = NP // NS


def _node_body(h_ref, wfct, wb1t, b1, wb2t, b2, wp, z_ref, b1h_ref, b2h_ref,
               pn_ref):
    hb = h_ref[...]
    z = jnp.dot(hb, wfct[...], preferred_element_type=jnp.float32)
    z_ref[...] = z
    b1h_ref[...] = jnp.dot(hb, wb1t[...],
                           preferred_element_type=jnp.float32) + b1[...][None, :]
    b2h_ref[...] = jnp.dot(hb, wb2t[...],
                           preferred_element_type=jnp.float32) + b2[...][None, :]
    pn_ref[...] = jnp.dot(z, wp[...], preferred_element_type=jnp.float32)


def _edge_body(e_ref, g_ref, wb3t, b3, u2, u2e, ej_ref, pe_ref, qe_ref):
    eb = e_ref[...]
    ej = jax.nn.relu(g_ref[...]
                     + jnp.dot(eb, wb3t[...], preferred_element_type=jnp.float32)
                     + b3[...][None, :]) + eb
    ej_ref[...] = ej
    pe_ref[...] = jnp.dot(ej, u2[...], preferred_element_type=jnp.float32)
    qe_ref[...] = jnp.dot(ej, u2e[...], preferred_element_type=jnp.float32)


def _tail_body(h1_ref, h2_ref, h_ref, wm1t, wct, bmix, gamma, beta, out_ref):
    hs = (jnp.dot(h1_ref[...], wm1t[...], preferred_element_type=jnp.float32)
          + jnp.dot(h2_ref[...], wct[...], preferred_element_type=jnp.float32)
          + 2.0 * bmix[...][None, :])
    mu = jnp.mean(hs, axis=-1, keepdims=True)
    var = jnp.mean((hs - mu) ** 2, axis=-1, keepdims=True)
    hn = (hs - mu) / jnp.sqrt(var + 1e-5) * gamma[...][None, :] + beta[...][None, :]
    out_ref[...] = jax.nn.relu(hn) + h_ref[...]


def _c1_body(src_h, dst_h, b1h_h, b2h_h, pn_h, g_h, t1_h, t2_h, t3_h, t4_h,
             si_v, di_v, g1_v, g2_v, pn_v, t1_v, t2_v, t3_v, t4_v,
             sg1, sg2, wg):
    cid = lax.axis_index("c")
    sid = lax.axis_index("s")
    wid = sid * NC + cid
    base = wid * EPW
    NCHUNK = EPW // CH
    pltpu.sync_copy(pn_h, pn_v)

    pltpu.sync_copy(src_h.at[pl.ds(base, CH)], si_v.at[0])
    pltpu.sync_copy(dst_h.at[pl.ds(base, CH)], di_v.at[0])
    pltpu.async_copy(b1h_h.at[si_v.at[0]], g1_v.at[0], sg1.at[0])
    pltpu.async_copy(b2h_h.at[di_v.at[0]], g2_v.at[0], sg2.at[0])

    def pair(m, carry):
        for b in (0, 1):
            k = 2 * m + b
            nb = 1 - b

            @pl.when(k + 1 < NCHUNK)
            def _():
                noff = base + (k + 1) * CH
                pltpu.sync_copy(src_h.at[pl.ds(noff, CH)], si_v.at[nb])
                pltpu.sync_copy(dst_h.at[pl.ds(noff, CH)], di_v.at[nb])

                @pl.when(k >= 1)
                def _():
                    pltpu.make_async_copy(g1_v.at[nb],
                                          g_h.at[pl.ds(base + (k - 1) * CH,
                                                       CH)],
                                          wg.at[nb]).wait()

                pltpu.async_copy(b1h_h.at[si_v.at[nb]], g1_v.at[nb],
                                 sg1.at[nb])
                pltpu.async_copy(b2h_h.at[di_v.at[nb]], g2_v.at[nb],
                                 sg2.at[nb])

            @pl.when(k < NCHUNK)
            def _():
              def tt(t, c):
                sl = pl.ds(t * L, L)
                tsl = pl.ds(k * CH + t * L, L)
                s16 = si_v[b, sl] * 4
                d16 = di_v[b, sl] * 4
                p1s = plsc.load_gather(pn_v, [s16])
                p3s = plsc.load_gather(pn_v, [s16 + 1])
                q1s = plsc.load_gather(pn_v, [s16 + 2])
                q3s = plsc.load_gather(pn_v, [s16 + 3])
                p1d = plsc.load_gather(pn_v, [d16])
                p3d = plsc.load_gather(pn_v, [d16 + 1])
                q1d = plsc.load_gather(pn_v, [d16 + 2])
                q3d = plsc.load_gather(pn_v, [d16 + 3])
                t1_v[tsl] = p1s + p3d
                t2_v[tsl] = p1d + p3s
                t3_v[tsl] = q1s + q3d
                t4_v[tsl] = q1d + q3s
                return c

              lax.fori_loop(0, CH // L, tt, 0)

              pltpu.make_async_copy(b1h_h.at[si_v.at[b]], g1_v.at[b],
                                    sg1.at[b]).wait()
              pltpu.make_async_copy(b2h_h.at[di_v.at[b]], g2_v.at[b],
                                    sg2.at[b]).wait()

              def row(i, c):
                for j in range(D // L):
                    sl = pl.ds(j * L, L)
                    g1_v[b, i, sl] = g1_v[b, i, sl] + g2_v[b, i, sl]
                return c

              lax.fori_loop(0, CH, row, 0)
              pltpu.async_copy(g1_v.at[b], g_h.at[pl.ds(base + k * CH, CH)],
                               wg.at[b])
        return carry

    lax.fori_loop(0, (NCHUNK + 1) // 2, pair, 0)
    pltpu.make_async_copy(g1_v.at[0],
                          g_h.at[pl.ds(base + (NCHUNK - 2) * CH, CH)],
                          wg.at[0]).wait()
    pltpu.make_async_copy(g1_v.at[1],
                          g_h.at[pl.ds(base + (NCHUNK - 1) * CH, CH)],
                          wg.at[1]).wait()
    pltpu.sync_copy(t1_v, t1_h.at[pl.ds(base, EPW)])
    pltpu.sync_copy(t2_v, t2_h.at[pl.ds(base, EPW)])
    pltpu.sync_copy(t3_v, t3_h.at[pl.ds(base, EPW)])
    pltpu.sync_copy(t4_v, t4_h.at[pl.ds(base, EPW)])


def _lrelu(x):
    return jnp.where(x >= 0.0, x, 0.01 * x)


def _c2_body(src_h, dst_h, t1_h, t2_h, t3_h, t4_h, pe_h, qe_h,
             exa_h, exr_h, exae_h, exare_h, sp_h,
             si_v, di_v, tb1, tb2, tb3, tb4, peb, qeb, ea_v, er_v, eae_v,
             eare_v, a0_v, a1_v, a2_v, a3_v, red_v, tmp_v, stage_s):
    cid = lax.axis_index("c")
    sid = lax.axis_index("s")
    wid = sid * NC + cid
    base = wid * EPW
    zero16 = jnp.zeros((L,), jnp.float32)

    def zacc(k, c):
        sl = pl.ds(k * L, L)
        a0_v[sl] = zero16
        a1_v[sl] = zero16
        a2_v[sl] = zero16
        a3_v[sl] = zero16
        return c

    lax.fori_loop(0, NP // L, zacc, 0)

    def chunk(k, carry):
        off = base + k * CH2
        sl_h = pl.ds(off, CH2)
        pltpu.sync_copy(src_h.at[sl_h], si_v)
        pltpu.sync_copy(dst_h.at[sl_h], di_v)
        pltpu.sync_copy(t1_h.at[sl_h], tb1)
        pltpu.sync_copy(t2_h.at[sl_h], tb2)
        pltpu.sync_copy(t3_h.at[sl_h], tb3)
        pltpu.sync_copy(t4_h.at[sl_h], tb4)
        pltpu.sync_copy(pe_h.at[sl_h], peb)
        pltpu.sync_copy(qe_h.at[sl_h], qeb)

        def tt(t, c):
            sl = pl.ds(t * L, L)
            s16 = si_v[sl]
            d16 = di_v[sl]
            pe = peb[sl]
            qe = qeb[sl]
            ea = jnp.exp(_lrelu(tb1[sl] + pe))
            er = jnp.exp(_lrelu(tb2[sl] + pe))
            eae = jnp.exp(_lrelu(tb3[sl] + qe))
            eare = jnp.exp(_lrelu(tb4[sl] + qe))
            ea_v[sl] = ea
            er_v[sl] = er
            eae_v[sl] = eae
            eare_v[sl] = eare
            plsc.addupdate_scatter(a0_v, [d16], ea)
            plsc.addupdate_scatter(a1_v, [s16], er)
            plsc.addupdate_scatter(a2_v, [d16], eae)
            plsc.addupdate_scatter(a3_v, [s16], eare)
            return c

        lax.fori_loop(0, CH2 // L, tt, 0)
        pltpu.sync_copy(ea_v, exa_h.at[sl_h])
        pltpu.sync_copy(er_v, exr_h.at[sl_h])
        pltpu.sync_copy(eae_v, exae_h.at[sl_h])
        pltpu.sync_copy(eare_v, exare_h.at[sl_h])
        return carry

    lax.fori_loop(0, EPW // CH2, chunk, 0)

    soff = sid * STR
    accs = [a0_v, a1_v, a2_v, a3_v]
    for r in range(4):
        pltpu.sync_copy(accs[r], stage_s.at[pl.ds(sid * NP, NP)])
        plsc.subcore_barrier()

        def zred(k, c):
            red_v[pl.ds(k * L, L)] = zero16
            return c

        lax.fori_loop(0, STR // L, zred, 0)

        def rt(t, c):
            pltpu.sync_copy(stage_s.at[pl.ds(t * NP + soff, STR)], tmp_v)

            def add(k, cc):
                sl = pl.ds(k * L, L)
                red_v[sl] = red_v[sl] + tmp_v[sl]
                return cc

            lax.fori_loop(0, STR // L, add, 0)
            return c

        lax.fori_loop(0, NS, rt, 0)
        pltpu.sync_copy(red_v, sp_h.at[pl.ds((cid * 4 + r) * NP + soff, STR)])
        plsc.subcore_barrier()


def _d_body(src_h, dst_h, z_h, ej_h, exa_h, exr_h, exae_h, exare_h, sp_h,
            h1_h, h2_h,
            si_v, di_v, r1_v, r2_v, ea_v, er_v, al_v, ar_v,
            sA_v, sB_v, tmp_v, zb_v, acc_s, sem1, sem2, sld):
    cid = lax.axis_index("c")
    sid = lax.axis_index("s")
    rA = 2 * cid

    pltpu.sync_copy(sp_h.at[pl.ds(rA * NP, NP)], sA_v)
    pltpu.sync_copy(sp_h.at[pl.ds((rA + 1) * NP, NP)], sB_v)

    def addpt(pt, c):
        pltpu.sync_copy(sp_h.at[pl.ds((4 + rA) * NP + pt * STR, STR)], tmp_v)

        def addA(k, cc):
            sl = pl.ds(pt * STR + k * L, L)
            sA_v[sl] = sA_v[sl] + tmp_v[pl.ds(k * L, L)]
            return cc

        lax.fori_loop(0, STR // L, addA, 0)
        pltpu.sync_copy(sp_h.at[pl.ds((4 + rA + 1) * NP + pt * STR, STR)],
                        tmp_v)

        def addB(k, cc):
            sl = pl.ds(pt * STR + k * L, L)
            sB_v[sl] = sB_v[sl] + tmp_v[pl.ds(k * L, L)]
            return cc

        lax.fori_loop(0, STR // L, addB, 0)
        return c

    lax.fori_loop(0, NS, addpt, 0)

    zero16 = jnp.zeros((L,), jnp.float32)

    def zrow(i, c):
        for j in range(D // L):
            zb_v[i, pl.ds(j * L, L)] = zero16
        return c

    lax.fori_loop(0, 8, zrow, 0)

    def zcp(rep, c):
        pltpu.sync_copy(zb_v, acc_s.at[pl.ds(sid * STR + rep * 8, 8)])
        return c

    lax.fori_loop(0, STR // 8, zcp, 0)
    plsc.subcore_barrier()

    base = sid * EPT
    NCHUNK = EPT // CH

    def issue_meta(k, b):
        sl_h = pl.ds(base + k * CH, CH)
        pltpu.async_copy(src_h.at[sl_h], si_v.at[b], sld.at[b])
        pltpu.async_copy(dst_h.at[sl_h], di_v.at[b], sld.at[b])

        @pl.when(cid == 0)
        def _():
            pltpu.async_copy(exa_h.at[sl_h], ea_v.at[b], sld.at[b])
            pltpu.async_copy(exr_h.at[sl_h], er_v.at[b], sld.at[b])

        @pl.when(cid == 1)
        def _():
            pltpu.async_copy(exae_h.at[sl_h], ea_v.at[b], sld.at[b])
            pltpu.async_copy(exare_h.at[sl_h], er_v.at[b], sld.at[b])

    def wait_meta(k, b):
        sl_h = pl.ds(base + k * CH, CH)
        pltpu.make_async_copy(src_h.at[sl_h], si_v.at[b], sld.at[b]).wait()
        pltpu.make_async_copy(dst_h.at[sl_h], di_v.at[b], sld.at[b]).wait()
        pltpu.make_async_copy(exa_h.at[sl_h], ea_v.at[b], sld.at[b]).wait()
        pltpu.make_async_copy(exr_h.at[sl_h], er_v.at[b], sld.at[b]).wait()

    issue_meta(0, 0)
    wait_meta(0, 0)

    def pair(m, carry):
        for b in (0, 1):
            k = 2 * m + b
            nb = 1 - b
            sl_h = pl.ds(base + k * CH, CH)

            @pl.when(cid == 0)
            def _():
                pltpu.async_copy(z_h.at[si_v.at[b]], r1_v, sem1)
                pltpu.async_copy(z_h.at[di_v.at[b]], r2_v, sem2)

            @pl.when(cid == 1)
            def _():
                pltpu.async_copy(ej_h.at[sl_h], r1_v, sem1)

            @pl.when(k + 1 < NCHUNK)
            def _():
                issue_meta(k + 1, nb)

            @pl.when(cid == 0)
            def _():
                pltpu.make_async_copy(z_h.at[si_v.at[b]], r1_v, sem1).wait()
                pltpu.make_async_copy(z_h.at[di_v.at[b]], r2_v, sem2).wait()

            @pl.when(cid == 1)
            def _():
                pltpu.make_async_copy(ej_h.at[sl_h], r1_v, sem1).wait()

            def tt(t, c):
                sl = pl.ds(t * L, L)
                s16 = si_v[b, sl]
                d16 = di_v[b, sl]
                den = plsc.load_gather(sA_v, [d16])
                al_v[sl] = ea_v[b, sl] / (den + 1e-9)
                den2 = plsc.load_gather(sB_v, [s16])
                ar_v[sl] = er_v[b, sl] / (den2 + 1e-9)
                return c

            lax.fori_loop(0, CH // L, tt, 0)

            @pl.when(cid == 0)
            def _():
                def grp(t, c):
                    al16 = al_v[pl.ds(t * L, L)]
                    ar16 = ar_v[pl.ds(t * L, L)]
                    for lane in range(L):
                        i = t * L + lane
                        a = al16[lane]
                        bb = ar16[lane]
                        for j in range(D // L):
                            sl = pl.ds(j * L, L)
                            r1_v[i, sl] = r1_v[i, sl] * a
                            r2_v[i, sl] = r2_v[i, sl] * bb
                    return c

                lax.fori_loop(0, CH // L, grp, 0)

            @pl.when(cid == 1)
            def _():
                def grp(t, c):
                    al16 = al_v[pl.ds(t * L, L)]
                    ar16 = ar_v[pl.ds(t * L, L)]
                    for lane in range(L):
                        i = t * L + lane
                        a = al16[lane]
                        bb = ar16[lane]
                        for j in range(D // L):
                            sl = pl.ds(j * L, L)
                            x = r1_v[i, sl]
                            r2_v[i, sl] = x * bb
                            r1_v[i, sl] = x * a
                    return c

                lax.fori_loop(0, CH // L, grp, 0)

            pltpu.sync_copy(r1_v, acc_s.at[di_v.at[b]], add=True)
            pltpu.sync_copy(r2_v, acc_s.at[si_v.at[b]], add=True)

            @pl.when(k + 1 < NCHUNK)
            def _():
                wait_meta(k + 1, nb)
        return carry

    lax.fori_loop(0, NCHUNK // 2, pair, 0)
    plsc.subcore_barrier()
    row_sl = pl.ds(sid * STR, STR)

    @pl.when(cid == 0)
    def _():
        pltpu.sync_copy(acc_s.at[row_sl], h1_h.at[row_sl])

    @pl.when(cid == 1)
    def _():
        pltpu.sync_copy(acc_s.at[row_sl], h2_h.at[row_sl])


def kernel(h, e, edge_index, W_fc, W_fce, W_attn, W_attn_e, W_B1, b_B1, W_B2,
           b_B2, W_B3, b_B3, W_mix, b_mix, ln_gamma, ln_beta):
    src = edge_index[0]
    dst = edge_index[1]

    w1, w2, w3 = W_attn[0, :D], W_attn[0, D:2 * D], W_attn[0, 2 * D:]
    w1e, w2e, w3e = W_attn_e[0, :D], W_attn_e[0, D:2 * D], W_attn_e[0, 2 * D:]
    u2 = (W_fce.T @ w2)[:, None]
    u2e = (W_fce.T @ w2e)[:, None]
    Wp = jnp.stack([w1, w3, w1e, w3e], axis=1)
    Wm1 = W_mix[:, :D]
    Wm2 = W_mix[:, D:]
    WcT = (Wm2 @ W_fce).T

    f32 = jnp.float32
    BN = 1000
    rep = lambda: pl.BlockSpec((D, D), lambda i: (0, 0))
    vec = lambda: pl.BlockSpec((D,), lambda i: (0,))

    z, b1h, b2h, pn = pl.pallas_call(
        _node_body,
        grid=(N // BN,),
        in_specs=[pl.BlockSpec((BN, D), lambda i: (i, 0)),
                  rep(), rep(), vec(), rep(), vec(),
                  pl.BlockSpec((D, 4), lambda i: (0, 0))],
        out_specs=[pl.BlockSpec((BN, D), lambda i: (i, 0))] * 3 + [
            pl.BlockSpec((BN, 4), lambda i: (i, 0))],
        out_shape=[jax.ShapeDtypeStruct((N, D), f32)] * 3 + [
            jax.ShapeDtypeStruct((N, 4), f32)],
    )(h, W_fc.T, W_B1.T, b_B1, W_B2.T, b_B2, Wp)

    mesh = plsc.VectorSubcoreMesh(core_axis_name="c", subcore_axis_name="s")
    c1 = pl.kernel(
        _c1_body,
        out_type=[jax.ShapeDtypeStruct((E, D), f32)] + [
            jax.ShapeDtypeStruct((E,), f32)] * 4,
        mesh=mesh,
        compiler_params=pltpu.CompilerParams(needs_layout_passes=False),
        scratch_types=[
            pltpu.VMEM((2, CH), jnp.int32), pltpu.VMEM((2, CH), jnp.int32),
            pltpu.VMEM((2, CH, D), f32), pltpu.VMEM((2, CH, D), f32),
            pltpu.VMEM((N * 4,), f32),
            pltpu.VMEM((EPW,), f32), pltpu.VMEM((EPW,), f32),
            pltpu.VMEM((EPW,), f32), pltpu.VMEM((EPW,), f32),
            pltpu.SemaphoreType.DMA((2,)), pltpu.SemaphoreType.DMA((2,)),
            pltpu.SemaphoreType.DMA((2,)),
        ],
    )
    g, t1, t2, t3, t4 = c1(src, dst, b1h, b2h, pn.reshape(N * 4))

    BE = 2000
    e_ji, pe, qe = pl.pallas_call(
        _edge_body,
        grid=(E // BE,),
        in_specs=[pl.BlockSpec((BE, D), lambda i: (i, 0)),
                  pl.BlockSpec((BE, D), lambda i: (i, 0)),
                  rep(), vec(),
                  pl.BlockSpec((D, 1), lambda i: (0, 0)),
                  pl.BlockSpec((D, 1), lambda i: (0, 0))],
        out_specs=[pl.BlockSpec((BE, D), lambda i: (i, 0)),
                   pl.BlockSpec((BE, 1), lambda i: (i, 0)),
                   pl.BlockSpec((BE, 1), lambda i: (i, 0))],
        out_shape=[jax.ShapeDtypeStruct((E, D), f32),
                   jax.ShapeDtypeStruct((E, 1), f32),
                   jax.ShapeDtypeStruct((E, 1), f32)],
    )(e, g, W_B3.T, b_B3, u2, u2e)
    pe1 = pe.reshape(E)
    qe1 = qe.reshape(E)

    c2 = pl.kernel(
        _c2_body,
        out_type=[jax.ShapeDtypeStruct((E,), f32)] * 4 + [
            jax.ShapeDtypeStruct((NC * 4 * NP,), f32)],
        mesh=mesh,
        compiler_params=pltpu.CompilerParams(needs_layout_passes=False),
        scratch_types=[
            pltpu.VMEM((CH2,), jnp.int32), pltpu.VMEM((CH2,), jnp.int32)] +
            [pltpu.VMEM((CH2,), f32)] * 10 +
            [pltpu.VMEM((NP,), f32)] * 4 +
            [pltpu.VMEM((STR,), f32), pltpu.VMEM((STR,), f32),
             pltpu.VMEM_SHARED((NS * NP,), f32)],
    )
    exa, exr, exae, exare, sp = c2(src, dst, t1, t2, t3, t4, pe1, qe1)

    dk = pl.kernel(
        _d_body,
        out_type=[jax.ShapeDtypeStruct((NP, D), f32),
                  jax.ShapeDtypeStruct((NP, D), f32)],
        mesh=mesh,
        compiler_params=pltpu.CompilerParams(needs_layout_passes=False),
        scratch_types=[
            pltpu.VMEM((2, CH), jnp.int32), pltpu.VMEM((2, CH), jnp.int32),
            pltpu.VMEM((CH, D), f32), pltpu.VMEM((CH, D), f32),
            pltpu.VMEM((2, CH), f32), pltpu.VMEM((2, CH), f32),
            pltpu.VMEM((CH,), f32), pltpu.VMEM((CH,), f32),
            pltpu.VMEM((NP,), f32), pltpu.VMEM((NP,), f32),
            pltpu.VMEM((STR,), f32),
            pltpu.VMEM((8, D), f32),
            pltpu.VMEM_SHARED((NP, D), f32),
            pltpu.SemaphoreType.DMA, pltpu.SemaphoreType.DMA,
            pltpu.SemaphoreType.DMA((2,)),
        ],
    )
    h1p, h2pp = dk(src, dst, z, e_ji, exa, exr, exae, exare, sp)
    h1 = h1p[:N]
    h2p = h2pp[:N]

    h_out = pl.pallas_call(
        _tail_body,
        grid=(N // BN,),
        in_specs=[pl.BlockSpec((BN, D), lambda i: (i, 0))] * 3 + [
            rep(), rep(), vec(), vec(), vec()],
        out_specs=pl.BlockSpec((BN, D), lambda i: (i, 0)),
        out_shape=jax.ShapeDtypeStruct((N, D), f32),
    )(h1, h2p, h, Wm1.T, WcT, b_mix, ln_gamma, ln_beta)

    return (h_out, e_ji)


# speedup vs baseline: 1.3346x; 1.0072x over previous
import jax
import jax.numpy as jnp
from jax import lax
from jax.experimental import pallas as pl
from jax.experimental.pallas import tpu as pltpu
from jax.experimental.pallas import tpu_sc as plsc

N = 10000
E = 320000
D = 128
NC = 2
NS = 16
NW = NC * NS
L = 16
NP = 10240
EPW = E // NW
EPT = E // NS
CH = 80
CH2 = 2000
CHD = 32
RPS = N // NS
STR = NP // NS


def _node_body(h_ref, wfct, wb1t, b1, wb2t, b2, wp, z_ref, b1h_ref, b2h_ref,
               pn_ref):
    hb = h_ref[...]
    z = jnp.dot(hb, wfct[...], preferred_element_type=jnp.float32)
    z_ref[...] = z
    b1h_ref[...] = jnp.dot(hb, wb1t[...],
                           preferred_element_type=jnp.float32) + b1[...][None, :]
    b2h_ref[...] = jnp.dot(hb, wb2t[...],
                           preferred_element_type=jnp.float32) + b2[...][None, :]
    pn_ref[...] = jnp.dot(z, wp[...], preferred_element_type=jnp.float32)


def _edge_body(e_ref, g_ref, wb3t, b3, u2, u2e, ej_ref, pe_ref, qe_ref):
    eb = e_ref[...]
    ej = jax.nn.relu(g_ref[...]
                     + jnp.dot(eb, wb3t[...], preferred_element_type=jnp.float32)
                     + b3[...][None, :]) + eb
    ej_ref[...] = ej
    pe_ref[...] = jnp.dot(ej, u2[...], preferred_element_type=jnp.float32)
    qe_ref[...] = jnp.dot(ej, u2e[...], preferred_element_type=jnp.float32)


def _tail_body(h1_ref, h2_ref, h_ref, wm1t, wct, bmix, gamma, beta, out_ref):
    hs = (jnp.dot(h1_ref[...], wm1t[...], preferred_element_type=jnp.float32)
          + jnp.dot(h2_ref[...], wct[...], preferred_element_type=jnp.float32)
          + 2.0 * bmix[...][None, :])
    mu = jnp.mean(hs, axis=-1, keepdims=True)
    var = jnp.mean((hs - mu) ** 2, axis=-1, keepdims=True)
    hn = (hs - mu) / jnp.sqrt(var + 1e-5) * gamma[...][None, :] + beta[...][None, :]
    out_ref[...] = jax.nn.relu(hn) + h_ref[...]


def _c1_body(src_h, dst_h, b1h_h, b2h_h, pn_h, g_h, t1_h, t2_h, t3_h, t4_h,
             si_v, di_v, g1_v, g2_v, pn_v, t1_v, t2_v, t3_v, t4_v,
             sg1, sg2, wg):
    cid = lax.axis_index("c")
    sid = lax.axis_index("s")
    wid = sid * NC + cid
    base = wid * EPW
    NCHUNK = EPW // CH
    pltpu.sync_copy(pn_h, pn_v)

    pltpu.sync_copy(src_h.at[pl.ds(base, CH)], si_v.at[0])
    pltpu.sync_copy(dst_h.at[pl.ds(base, CH)], di_v.at[0])
    pltpu.async_copy(b1h_h.at[si_v.at[0]], g1_v.at[0], sg1.at[0])
    pltpu.async_copy(b2h_h.at[di_v.at[0]], g2_v.at[0], sg2.at[0])

    def pair(m, carry):
        for b in (0, 1):
            k = 2 * m + b
            nb = 1 - b

            @pl.when(k + 1 < NCHUNK)
            def _():
                noff = base + (k + 1) * CH
                pltpu.sync_copy(src_h.at[pl.ds(noff, CH)], si_v.at[nb])
                pltpu.sync_copy(dst_h.at[pl.ds(noff, CH)], di_v.at[nb])

                @pl.when(k >= 1)
                def _():
                    pltpu.make_async_copy(g1_v.at[nb],
                                          g_h.at[pl.ds(base + (k - 1) * CH,
                                                       CH)],
                                          wg.at[nb]).wait()

                pltpu.async_copy(b1h_h.at[si_v.at[nb]], g1_v.at[nb],
                                 sg1.at[nb])
                pltpu.async_copy(b2h_h.at[di_v.at[nb]], g2_v.at[nb],
                                 sg2.at[nb])

            @pl.when(k < NCHUNK)
            def _():
              def tt(t, c):
                sl = pl.ds(t * L, L)
                tsl = pl.ds(k * CH + t * L, L)
                s16 = si_v[b, sl] * 4
                d16 = di_v[b, sl] * 4
                p1s = plsc.load_gather(pn_v, [s16])
                p3s = plsc.load_gather(pn_v, [s16 + 1])
                q1s = plsc.load_gather(pn_v, [s16 + 2])
                q3s = plsc.load_gather(pn_v, [s16 + 3])
                p1d = plsc.load_gather(pn_v, [d16])
                p3d = plsc.load_gather(pn_v, [d16 + 1])
                q1d = plsc.load_gather(pn_v, [d16 + 2])
                q3d = plsc.load_gather(pn_v, [d16 + 3])
                t1_v[tsl] = p1s + p3d
                t2_v[tsl] = p1d + p3s
                t3_v[tsl] = q1s + q3d
                t4_v[tsl] = q1d + q3s
                return c

              lax.fori_loop(0, CH // L, tt, 0)

              pltpu.make_async_copy(b1h_h.at[si_v.at[b]], g1_v.at[b],
                                    sg1.at[b]).wait()
              pltpu.make_async_copy(b2h_h.at[di_v.at[b]], g2_v.at[b],
                                    sg2.at[b]).wait()

              def row(i, c):
                for j in range(D // L):
                    sl = pl.ds(j * L, L)
                    g1_v[b, i, sl] = g1_v[b, i, sl] + g2_v[b, i, sl]
                return c

              lax.fori_loop(0, CH, row, 0)
              pltpu.async_copy(g1_v.at[b], g_h.at[pl.ds(base + k * CH, CH)],
                               wg.at[b])
        return carry

    lax.fori_loop(0, (NCHUNK + 1) // 2, pair, 0)
    pltpu.make_async_copy(g1_v.at[0],
                          g_h.at[pl.ds(base + (NCHUNK - 2) * CH, CH)],
                          wg.at[0]).wait()
    pltpu.make_async_copy(g1_v.at[1],
                          g_h.at[pl.ds(base + (NCHUNK - 1) * CH, CH)],
                          wg.at[1]).wait()
    pltpu.sync_copy(t1_v, t1_h.at[pl.ds(base, EPW)])
    pltpu.sync_copy(t2_v, t2_h.at[pl.ds(base, EPW)])
    pltpu.sync_copy(t3_v, t3_h.at[pl.ds(base, EPW)])
    pltpu.sync_copy(t4_v, t4_h.at[pl.ds(base, EPW)])


def _lrelu(x):
    return jnp.where(x >= 0.0, x, 0.01 * x)


def _c2_body(src_h, dst_h, t1_h, t2_h, t3_h, t4_h, pe_h, qe_h,
             exa_h, exr_h, exae_h, exare_h, sp_h,
             si_v, di_v, tb1, tb2, tb3, tb4, peb, qeb, ea_v, er_v, eae_v,
             eare_v, a0_v, a1_v, a2_v, a3_v, red_v, tmp_v, stage_s):
    cid = lax.axis_index("c")
    sid = lax.axis_index("s")
    wid = sid * NC + cid
    base = wid * EPW
    zero16 = jnp.zeros((L,), jnp.float32)

    def zacc(k, c):
        sl = pl.ds(k * L, L)
        a0_v[sl] = zero16
        a1_v[sl] = zero16
        a2_v[sl] = zero16
        a3_v[sl] = zero16
        return c

    lax.fori_loop(0, NP // L, zacc, 0)

    def chunk(k, carry):
        off = base + k * CH2
        sl_h = pl.ds(off, CH2)
        pltpu.sync_copy(src_h.at[sl_h], si_v)
        pltpu.sync_copy(dst_h.at[sl_h], di_v)
        pltpu.sync_copy(t1_h.at[sl_h], tb1)
        pltpu.sync_copy(t2_h.at[sl_h], tb2)
        pltpu.sync_copy(t3_h.at[sl_h], tb3)
        pltpu.sync_copy(t4_h.at[sl_h], tb4)
        pltpu.sync_copy(pe_h.at[sl_h], peb)
        pltpu.sync_copy(qe_h.at[sl_h], qeb)

        def tt(t, c):
            sl = pl.ds(t * L, L)
            s16 = si_v[sl]
            d16 = di_v[sl]
            pe = peb[sl]
            qe = qeb[sl]
            ea = jnp.exp(_lrelu(tb1[sl] + pe))
            er = jnp.exp(_lrelu(tb2[sl] + pe))
            eae = jnp.exp(_lrelu(tb3[sl] + qe))
            eare = jnp.exp(_lrelu(tb4[sl] + qe))
            ea_v[sl] = ea
            er_v[sl] = er
            eae_v[sl] = eae
            eare_v[sl] = eare
            plsc.addupdate_scatter(a0_v, [d16], ea)
            plsc.addupdate_scatter(a1_v, [s16], er)
            plsc.addupdate_scatter(a2_v, [d16], eae)
            plsc.addupdate_scatter(a3_v, [s16], eare)
            return c

        lax.fori_loop(0, CH2 // L, tt, 0)
        pltpu.sync_copy(ea_v, exa_h.at[sl_h])
        pltpu.sync_copy(er_v, exr_h.at[sl_h])
        pltpu.sync_copy(eae_v, exae_h.at[sl_h])
        pltpu.sync_copy(eare_v, exare_h.at[sl_h])
        return carry

    lax.fori_loop(0, EPW // CH2, chunk, 0)

    soff = sid * STR
    accs = [a0_v, a1_v, a2_v, a3_v]
    for r in range(4):
        pltpu.sync_copy(accs[r], stage_s.at[pl.ds(sid * NP, NP)])
        plsc.subcore_barrier()

        def zred(k, c):
            red_v[pl.ds(k * L, L)] = zero16
            return c

        lax.fori_loop(0, STR // L, zred, 0)

        def rt(t, c):
            pltpu.sync_copy(stage_s.at[pl.ds(t * NP + soff, STR)], tmp_v)

            def add(k, cc):
                sl = pl.ds(k * L, L)
                red_v[sl] = red_v[sl] + tmp_v[sl]
                return cc

            lax.fori_loop(0, STR // L, add, 0)
            return c

        lax.fori_loop(0, NS, rt, 0)
        pltpu.sync_copy(red_v, sp_h.at[pl.ds((cid * 4 + r) * NP + soff, STR)])
        plsc.subcore_barrier()


def _d_body(src_h, dst_h, z_h, ej_h, exa_h, exr_h, exae_h, exare_h, sp_h,
            h1_h, h2_h,
            si_v, di_v, r1_v, r2_v, ea_v, er_v, al_v, ar_v,
            sA_v, sB_v, tmp_v, zb_v, acc_s, sem1, sem2, sld):
    cid = lax.axis_index("c")
    sid = lax.axis_index("s")
    rA = 2 * cid

    pltpu.sync_copy(sp_h.at[pl.ds(rA * NP, NP)], sA_v)
    pltpu.sync_copy(sp_h.at[pl.ds((rA + 1) * NP, NP)], sB_v)

    def addpt(pt, c):
        pltpu.sync_copy(sp_h.at[pl.ds((4 + rA) * NP + pt * STR, STR)], tmp_v)

        def addA(k, cc):
            sl = pl.ds(pt * STR + k * L, L)
            sA_v[sl] = sA_v[sl] + tmp_v[pl.ds(k * L, L)]
            return cc

        lax.fori_loop(0, STR // L, addA, 0)
        pltpu.sync_copy(sp_h.at[pl.ds((4 + rA + 1) * NP + pt * STR, STR)],
                        tmp_v)

        def addB(k, cc):
            sl = pl.ds(pt * STR + k * L, L)
            sB_v[sl] = sB_v[sl] + tmp_v[pl.ds(k * L, L)]
            return cc

        lax.fori_loop(0, STR // L, addB, 0)
        return c

    lax.fori_loop(0, NS, addpt, 0)

    zero16 = jnp.zeros((L,), jnp.float32)

    def zrow(i, c):
        for j in range(D // L):
            zb_v[i, pl.ds(j * L, L)] = zero16
        return c

    lax.fori_loop(0, 8, zrow, 0)

    def zcp(rep, c):
        pltpu.sync_copy(zb_v, acc_s.at[pl.ds(sid * STR + rep * 8, 8)])
        return c

    lax.fori_loop(0, STR // 8, zcp, 0)
    plsc.subcore_barrier()

    base = sid * EPT
    NCHUNK = EPT // CH

    def issue_meta(k, b):
        sl_h = pl.ds(base + k * CH, CH)
        pltpu.async_copy(src_h.at[sl_h], si_v.at[b], sld.at[b])
        pltpu.async_copy(dst_h.at[sl_h], di_v.at[b], sld.at[b])

        @pl.when(cid == 0)
        def _():
            pltpu.async_copy(exa_h.at[sl_h], ea_v.at[b], sld.at[b])
            pltpu.async_copy(exr_h.at[sl_h], er_v.at[b], sld.at[b])

        @pl.when(cid == 1)
        def _():
            pltpu.async_copy(exae_h.at[sl_h], ea_v.at[b], sld.at[b])
            pltpu.async_copy(exare_h.at[sl_h], er_v.at[b], sld.at[b])

    def wait_meta(k, b):
        sl_h = pl.ds(base + k * CH, CH)
        pltpu.make_async_copy(src_h.at[sl_h], si_v.at[b], sld.at[b]).wait()
        pltpu.make_async_copy(dst_h.at[sl_h], di_v.at[b], sld.at[b]).wait()
        pltpu.make_async_copy(exa_h.at[sl_h], ea_v.at[b], sld.at[b]).wait()
        pltpu.make_async_copy(exr_h.at[sl_h], er_v.at[b], sld.at[b]).wait()

    issue_meta(0, 0)
    wait_meta(0, 0)

    def pair(m, carry):
        for b in (0, 1):
            k = 2 * m + b
            nb = 1 - b
            sl_h = pl.ds(base + k * CH, CH)

            @pl.when(cid == 0)
            def _():
                pltpu.async_copy(z_h.at[si_v.at[b]], r1_v, sem1)
                pltpu.async_copy(z_h.at[di_v.at[b]], r2_v, sem2)

            @pl.when(cid == 1)
            def _():
                pltpu.async_copy(ej_h.at[sl_h], r1_v, sem1)

            @pl.when(k + 1 < NCHUNK)
            def _():
                issue_meta(k + 1, nb)

            @pl.when(cid == 0)
            def _():
                pltpu.make_async_copy(z_h.at[si_v.at[b]], r1_v, sem1).wait()
                pltpu.make_async_copy(z_h.at[di_v.at[b]], r2_v, sem2).wait()

            @pl.when(cid == 1)
            def _():
                pltpu.make_async_copy(ej_h.at[sl_h], r1_v, sem1).wait()

            def tt(t, c):
                sl = pl.ds(t * L, L)
                s16 = si_v[b, sl]
                d16 = di_v[b, sl]
                den = plsc.load_gather(sA_v, [d16])
                al_v[sl] = ea_v[b, sl] / (den + 1e-9)
                den2 = plsc.load_gather(sB_v, [s16])
                ar_v[sl] = er_v[b, sl] / (den2 + 1e-9)
                return c

            lax.fori_loop(0, CH // L, tt, 0)

            @pl.when(cid == 0)
            def _():
                def grp(t, c):
                    al16 = al_v[pl.ds(t * L, L)]
                    ar16 = ar_v[pl.ds(t * L, L)]
                    for lane in range(L):
                        i = t * L + lane
                        a = al16[lane]
                        bb = ar16[lane]
                        for j in range(D // L):
                            sl = pl.ds(j * L, L)
                            r1_v[i, sl] = r1_v[i, sl] * a
                            r2_v[i, sl] = r2_v[i, sl] * bb
                    return c

                lax.fori_loop(0, CH // L, grp, 0)

            @pl.when(cid == 1)
            def _():
                def grp(t, c):
                    al16 = al_v[pl.ds(t * L, L)]
                    ar16 = ar_v[pl.ds(t * L, L)]
                    for lane in range(L):
                        i = t * L + lane
                        a = al16[lane]
                        bb = ar16[lane]
                        for j in range(D // L):
                            sl = pl.ds(j * L, L)
                            x = r1_v[i, sl]
                            r2_v[i, sl] = x * bb
                            r1_v[i, sl] = x * a
                    return c

                lax.fori_loop(0, CH // L, grp, 0)

            sc1 = pltpu.async_copy(r1_v, acc_s.at[di_v.at[b]], sem1,
                                   add=True)
            sc2 = pltpu.async_copy(r2_v, acc_s.at[si_v.at[b]], sem2,
                                   add=True)
            sc1.wait()
            sc2.wait()

            @pl.when(k + 1 < NCHUNK)
            def _():
                wait_meta(k + 1, nb)
        return carry

    lax.fori_loop(0, NCHUNK // 2, pair, 0)
    plsc.subcore_barrier()
    row_sl = pl.ds(sid * STR, STR)

    @pl.when(cid == 0)
    def _():
        pltpu.sync_copy(acc_s.at[row_sl], h1_h.at[row_sl])

    @pl.when(cid == 1)
    def _():
        pltpu.sync_copy(acc_s.at[row_sl], h2_h.at[row_sl])


def kernel(h, e, edge_index, W_fc, W_fce, W_attn, W_attn_e, W_B1, b_B1, W_B2,
           b_B2, W_B3, b_B3, W_mix, b_mix, ln_gamma, ln_beta):
    src = edge_index[0]
    dst = edge_index[1]

    w1, w2, w3 = W_attn[0, :D], W_attn[0, D:2 * D], W_attn[0, 2 * D:]
    w1e, w2e, w3e = W_attn_e[0, :D], W_attn_e[0, D:2 * D], W_attn_e[0, 2 * D:]
    u2 = (W_fce.T @ w2)[:, None]
    u2e = (W_fce.T @ w2e)[:, None]
    Wp = jnp.stack([w1, w3, w1e, w3e], axis=1)
    Wm1 = W_mix[:, :D]
    Wm2 = W_mix[:, D:]
    WcT = (Wm2 @ W_fce).T

    f32 = jnp.float32
    BN = 1000
    rep = lambda: pl.BlockSpec((D, D), lambda i: (0, 0))
    vec = lambda: pl.BlockSpec((D,), lambda i: (0,))

    z, b1h, b2h, pn = pl.pallas_call(
        _node_body,
        grid=(N // BN,),
        in_specs=[pl.BlockSpec((BN, D), lambda i: (i, 0)),
                  rep(), rep(), vec(), rep(), vec(),
                  pl.BlockSpec((D, 4), lambda i: (0, 0))],
        out_specs=[pl.BlockSpec((BN, D), lambda i: (i, 0))] * 3 + [
            pl.BlockSpec((BN, 4), lambda i: (i, 0))],
        out_shape=[jax.ShapeDtypeStruct((N, D), f32)] * 3 + [
            jax.ShapeDtypeStruct((N, 4), f32)],
    )(h, W_fc.T, W_B1.T, b_B1, W_B2.T, b_B2, Wp)

    mesh = plsc.VectorSubcoreMesh(core_axis_name="c", subcore_axis_name="s")
    c1 = pl.kernel(
        _c1_body,
        out_type=[jax.ShapeDtypeStruct((E, D), f32)] + [
            jax.ShapeDtypeStruct((E,), f32)] * 4,
        mesh=mesh,
        compiler_params=pltpu.CompilerParams(needs_layout_passes=False),
        scratch_types=[
            pltpu.VMEM((2, CH), jnp.int32), pltpu.VMEM((2, CH), jnp.int32),
            pltpu.VMEM((2, CH, D), f32), pltpu.VMEM((2, CH, D), f32),
            pltpu.VMEM((N * 4,), f32),
            pltpu.VMEM((EPW,), f32), pltpu.VMEM((EPW,), f32),
            pltpu.VMEM((EPW,), f32), pltpu.VMEM((EPW,), f32),
            pltpu.SemaphoreType.DMA((2,)), pltpu.SemaphoreType.DMA((2,)),
            pltpu.SemaphoreType.DMA((2,)),
        ],
    )
    g, t1, t2, t3, t4 = c1(src, dst, b1h, b2h, pn.reshape(N * 4))

    BE = 2000
    e_ji, pe, qe = pl.pallas_call(
        _edge_body,
        grid=(E // BE,),
        in_specs=[pl.BlockSpec((BE, D), lambda i: (i, 0)),
                  pl.BlockSpec((BE, D), lambda i: (i, 0)),
                  rep(), vec(),
                  pl.BlockSpec((D, 1), lambda i: (0, 0)),
                  pl.BlockSpec((D, 1), lambda i: (0, 0))],
        out_specs=[pl.BlockSpec((BE, D), lambda i: (i, 0)),
                   pl.BlockSpec((BE, 1), lambda i: (i, 0)),
                   pl.BlockSpec((BE, 1), lambda i: (i, 0))],
        out_shape=[jax.ShapeDtypeStruct((E, D), f32),
                   jax.ShapeDtypeStruct((E, 1), f32),
                   jax.ShapeDtypeStruct((E, 1), f32)],
    )(e, g, W_B3.T, b_B3, u2, u2e)
    pe1 = pe.reshape(E)
    qe1 = qe.reshape(E)

    c2 = pl.kernel(
        _c2_body,
        out_type=[jax.ShapeDtypeStruct((E,), f32)] * 4 + [
            jax.ShapeDtypeStruct((NC * 4 * NP,), f32)],
        mesh=mesh,
        compiler_params=pltpu.CompilerParams(needs_layout_passes=False),
        scratch_types=[
            pltpu.VMEM((CH2,), jnp.int32), pltpu.VMEM((CH2,), jnp.int32)] +
            [pltpu.VMEM((CH2,), f32)] * 10 +
            [pltpu.VMEM((NP,), f32)] * 4 +
            [pltpu.VMEM((STR,), f32), pltpu.VMEM((STR,), f32),
             pltpu.VMEM_SHARED((NS * NP,), f32)],
    )
    exa, exr, exae, exare, sp = c2(src, dst, t1, t2, t3, t4, pe1, qe1)

    dk = pl.kernel(
        _d_body,
        out_type=[jax.ShapeDtypeStruct((NP, D), f32),
                  jax.ShapeDtypeStruct((NP, D), f32)],
        mesh=mesh,
        compiler_params=pltpu.CompilerParams(needs_layout_passes=False),
        scratch_types=[
            pltpu.VMEM((2, CH), jnp.int32), pltpu.VMEM((2, CH), jnp.int32),
            pltpu.VMEM((CH, D), f32), pltpu.VMEM((CH, D), f32),
            pltpu.VMEM((2, CH), f32), pltpu.VMEM((2, CH), f32),
            pltpu.VMEM((CH,), f32), pltpu.VMEM((CH,), f32),
            pltpu.VMEM((NP,), f32), pltpu.VMEM((NP,), f32),
            pltpu.VMEM((STR,), f32),
            pltpu.VMEM((8, D), f32),
            pltpu.VMEM_SHARED((NP, D), f32),
            pltpu.SemaphoreType.DMA, pltpu.SemaphoreType.DMA,
            pltpu.SemaphoreType.DMA((2,)),
        ],
    )
    h1p, h2pp = dk(src, dst, z, e_ji, exa, exr, exae, exare, sp)
    h1 = h1p[:N]
    h2p = h2pp[:N]

    h_out = pl.pallas_call(
        _tail_body,
        grid=(N // BN,),
        in_specs=[pl.BlockSpec((BN, D), lambda i: (i, 0))] * 3 + [
            rep(), rep(), vec(), vec(), vec()],
        out_specs=pl.BlockSpec((BN, D), lambda i: (i, 0)),
        out_shape=jax.ShapeDtypeStruct((N, D), f32),
    )(h1, h2p, h, Wm1.T, WcT, b_mix, ln_gamma, ln_beta)

    return (h_out, e_ji)
